# baseline (speedup 1.0000x reference)
import functools

import jax
import jax.numpy as jnp
from jax import lax
from jax.experimental import pallas as pl
from jax.experimental.pallas import tpu as pltpu
from jax.experimental.pallas import tpu_sc as plsc

D_MODEL = 64
SCALE = 8.0

NC = 2
NS = 16
L = 16
NW = NC * NS

ROWS_PER_GATHER = 128
GATHERS_PER_CHUNK = 8
CHUNK = ROWS_PER_GATHER * GATHERS_PER_CHUNK


@functools.cache
def _build(B):
    assert B % (NW * CHUNK) == 0
    b_per_w = B // NW
    n_chunks = b_per_w // CHUNK
    mesh = plsc.VectorSubcoreMesh(core_axis_name="c", subcore_axis_name="s")

    @functools.partial(
        pl.kernel,
        out_type=jax.ShapeDtypeStruct((B, D_MODEL), jnp.float32),
        mesh=mesh,
        compiler_params=pltpu.CompilerParams(use_tc_tiling_on_sc=False),
        scratch_types=[
            pltpu.VMEM((GATHERS_PER_CHUNK, ROWS_PER_GATHER), jnp.int32),
            pltpu.VMEM((CHUNK, D_MODEL), jnp.float32),
            pltpu.SemaphoreType.DMA,
        ],
    )
    def emb_kernel(x_hbm, table_hbm, out_hbm, idx_v, rows_v, sem):
        wid = lax.axis_index("s") * NC + lax.axis_index("c")
        row_base = wid * b_per_w

        @pl.loop(0, n_chunks)
        def chunk(g):
            off = row_base + g * CHUNK
            grp = pl.multiple_of(off // ROWS_PER_GATHER, 8)
            pltpu.sync_copy(
                x_hbm.at[pl.ds(grp, GATHERS_PER_CHUNK)],
                idx_v,
            )
            copies = [
                pltpu.async_copy(
                    table_hbm.at[idx_v.at[j]],
                    rows_v.at[pl.ds(j * ROWS_PER_GATHER, ROWS_PER_GATHER)],
                    sem,
                )
                for j in range(GATHERS_PER_CHUNK)
            ]
            for c in copies:
                c.wait()

            @pl.loop(0, CHUNK)
            def scale_row(r):
                for c in range(D_MODEL // L):
                    v = rows_v[r, pl.ds(c * L, L)]
                    rows_v[r, pl.ds(c * L, L)] = v * SCALE

            pltpu.sync_copy(rows_v, out_hbm.at[pl.ds(off, CHUNK)])

    return emb_kernel


def kernel(x, table):
    B0, B1 = x.shape
    B = B0 * B1
    x2 = x.reshape(B // ROWS_PER_GATHER, ROWS_PER_GATHER)
    out = _build(B)(x2, table)
    return out.reshape(B0, B1, D_MODEL)


# speedup vs baseline: 1.1129x; 1.1129x over previous
import functools

import jax
import jax.numpy as jnp
from jax import lax
from jax.experimental import pallas as pl
from jax.experimental.pallas import tpu as pltpu
from jax.experimental.pallas import tpu_sc as plsc

D_MODEL = 64
SCALE = 8.0

NC = 2
NS = 16
L = 16
NW = NC * NS

G_ROWS = 128
GPC = 4
CHUNK = G_ROWS * GPC


@functools.cache
def _build(B):
    assert B % (NW * CHUNK) == 0
    b_per_w = B // NW
    n_chunks = b_per_w // CHUNK
    n_groups = b_per_w // G_ROWS
    mesh = plsc.VectorSubcoreMesh(core_axis_name="c", subcore_axis_name="s")

    @functools.partial(
        pl.kernel,
        out_type=jax.ShapeDtypeStruct((B, D_MODEL), jnp.float32),
        mesh=mesh,
        compiler_params=pltpu.CompilerParams(use_tc_tiling_on_sc=False),
        scratch_types=[
            pltpu.VMEM((n_groups, G_ROWS), jnp.int32),
            pltpu.VMEM((CHUNK, D_MODEL), jnp.float32),
            pltpu.VMEM((CHUNK, D_MODEL), jnp.float32),
            pltpu.SemaphoreType.DMA,
            pltpu.SemaphoreType.DMA,
            pltpu.SemaphoreType.DMA,
            pltpu.SemaphoreType.DMA,
        ],
    )
    def emb_kernel(x_hbm, table_hbm, out_hbm, idx_v, rows0, rows1,
                   sem_g0, sem_g1, sem_s0, sem_s1):
        wid = lax.axis_index("s") * NC + lax.axis_index("c")
        row_base = wid * b_per_w
        rows = (rows0, rows1)
        sem_g = (sem_g0, sem_g1)
        sem_s = (sem_s0, sem_s1)

        grp0 = pl.multiple_of(row_base // G_ROWS, 8)
        pltpu.sync_copy(x_hbm.at[pl.ds(grp0, n_groups)], idx_v)

        def fire_gather(t, b):
            for j in range(GPC):
                pltpu.async_copy(
                    table_hbm.at[idx_v.at[t * GPC + j]],
                    rows[b].at[pl.ds(j * G_ROWS, G_ROWS)],
                    sem_g[b],
                )

        def wait_gather(t, b):
            for j in range(GPC):
                pltpu.make_async_copy(
                    table_hbm.at[idx_v.at[t * GPC + j]],
                    rows[b].at[pl.ds(j * G_ROWS, G_ROWS)],
                    sem_g[b],
                ).wait()

        def out_slice(t):
            return out_hbm.at[pl.ds(row_base + t * CHUNK, CHUNK)]

        fire_gather(0, 0)

        @pl.loop(0, n_chunks, step=2)
        def chunk(g):
            for tb in range(2):
                t = g + tb
                other = 1 - tb

                @pl.when(t >= 1)
                def _wait_prev_store():
                    pltpu.make_async_copy(
                        rows[other], out_slice(t - 1), sem_s[other]
                    ).wait()

                @pl.when(t + 1 < n_chunks)
                def _fire_next_gather():
                    fire_gather(t + 1, other)

                wait_gather(t, tb)

                @plsc.parallel_loop(0, CHUNK, unroll=4)
                def scale_row(r):
                    for c in range(D_MODEL // L):
                        v = rows[tb][r, pl.ds(c * L, L)]
                        rows[tb][r, pl.ds(c * L, L)] = v * SCALE

                pltpu.async_copy(rows[tb], out_slice(t), sem_s[tb])

        pltpu.make_async_copy(rows[1], out_slice(n_chunks - 1), sem_s[1]).wait()

    return emb_kernel


def kernel(x, table):
    B0, B1 = x.shape
    B = B0 * B1
    x2 = x.reshape(B // G_ROWS, G_ROWS)
    out = _build(B)(x2, table)
    return out.reshape(B0, B1, D_MODEL)
